# Initial kernel scaffold; baseline (speedup 1.0000x reference)
import jax
import jax.numpy as jnp
from jax.experimental import pallas as pl


def kernel(x, edge_index):
    raise NotImplementedError("write your pallas kernel here")


# speedup vs baseline: 23.9755x; 23.9755x over previous
import functools

import jax
import jax.numpy as jnp
from jax import lax
from jax.experimental import pallas as pl
from jax.experimental.pallas import tpu as pltpu
from jax.experimental.pallas import tpu_sc as plsc

N = 10000
E = 320000
D = 128
NC = 2
NS = 16
NW = NC * NS
EPT = E // NW
K = 80
NCH = EPT // K
SEG = 640
N_PAD = NS * SEG

_MESH = plsc.VectorSubcoreMesh(core_axis_name="c", subcore_axis_name="s")


def _deg_body(dst3_hbm, out_hbm, dst_idx, buf, acc):
    c = lax.axis_index("c")
    s = lax.axis_index("s")
    wid = s * NC + c
    pltpu.sync_copy(dst3_hbm.at[wid], dst_idx)

    @pl.loop(0, SEG // 16)
    def _(i):
        buf[pl.ds(i * 16, 16)] = jnp.zeros((16,), jnp.float32)

    pltpu.sync_copy(buf, acc.at[pl.ds(s * SEG, SEG)])

    @pl.loop(0, 8)
    def _(i):
        buf[pl.ds(i * 16, 16)] = jnp.ones((16,), jnp.float32)

    plsc.subcore_barrier()

    @pl.loop(0, NCH)
    def _(j):
        pltpu.sync_copy(buf.at[pl.ds(0, K)], acc.at[dst_idx.at[j]], add=True)

    plsc.subcore_barrier()
    pltpu.sync_copy(acc.at[pl.ds(s * SEG, SEG)],
                    out_hbm.at[c].at[pl.ds(s * SEG, SEG)])


_deg_call = functools.partial(
    pl.kernel,
    out_type=jax.ShapeDtypeStruct((NC, N_PAD), jnp.float32),
    mesh=_MESH,
    scratch_types=[
        pltpu.VMEM((NCH, K), jnp.int32),
        pltpu.VMEM((SEG,), jnp.float32),
        pltpu.VMEM_SHARED((N_PAD,), jnp.float32),
    ],
)(_deg_body)


def _main_body(xs_hbm, src1_hbm, dst4_hbm, out_hbm,
               src_idx, dbuf0, dbuf1, buf0, buf1,
               sem0, sem1, dsem0, dsem1, acc):
    c = lax.axis_index("c")
    s = lax.axis_index("s")
    wid = s * NC + c
    pltpu.sync_copy(src1_hbm.at[pl.ds(wid * EPT, EPT)], src_idx)

    @pl.loop(0, K)
    def _(i):
        for jj in range(D // 16):
            buf0[i, pl.ds(jj * 16, 16)] = jnp.zeros((16,), jnp.float32)

    for k in range(SEG // K):
        pltpu.sync_copy(buf0, acc.at[pl.ds(s * SEG + k * K, K)])

    plsc.subcore_barrier()

    bufs = ((buf0, sem0, dbuf0, dsem0), (buf1, sem1, dbuf1, dsem1))

    def start(jj, buf, sem, dbuf, dsem):
        pltpu.async_copy(xs_hbm.at[src_idx.at[pl.ds(jj * K, K)]], buf, sem)
        pltpu.async_copy(dst4_hbm.at[wid * NCH + jj], dbuf, dsem)

    def finish(jj, buf, sem, dbuf, dsem):
        pltpu.make_async_copy(
            xs_hbm.at[src_idx.at[pl.ds(jj * K, K)]], buf, sem).wait()
        pltpu.make_async_copy(dst4_hbm.at[wid * NCH + jj], dbuf, dsem).wait()
        pltpu.sync_copy(buf, acc.at[dbuf.at[0]], add=True)

    for b in range(2):
        start(b, *bufs[b])

    @pl.loop(0, NCH - 1, step=2)
    def _(j):
        for b in range(2):
            jj = j + b
            buf, sem, dbuf, dsem = bufs[b]
            finish(jj, buf, sem, dbuf, dsem)

            @pl.when(jj + 2 < NCH)
            def _():
                start(jj + 2, buf, sem, dbuf, dsem)

    finish(NCH - 1, *bufs[(NCH - 1) % 2])

    plsc.subcore_barrier()
    pltpu.sync_copy(acc.at[pl.ds(s * SEG, SEG)],
                    out_hbm.at[c].at[pl.ds(s * SEG, SEG)])


_main_call = functools.partial(
    pl.kernel,
    out_type=jax.ShapeDtypeStruct((NC, N_PAD, D), jnp.float32),
    mesh=_MESH,
    scratch_types=[
        pltpu.VMEM((EPT,), jnp.int32),
        pltpu.VMEM((1, K), jnp.int32),
        pltpu.VMEM((1, K), jnp.int32),
        pltpu.VMEM((K, D), jnp.float32),
        pltpu.VMEM((K, D), jnp.float32),
        pltpu.SemaphoreType.DMA,
        pltpu.SemaphoreType.DMA,
        pltpu.SemaphoreType.DMA,
        pltpu.SemaphoreType.DMA,
        pltpu.VMEM_SHARED((N_PAD, D), jnp.float32),
    ],
)(_main_body)


def _prescale_body(x_ref, degp_ref, xs_ref):
    deg = degp_ref[0] + degp_ref[1]
    dinv = jnp.where(deg > 0.0, lax.rsqrt(deg), 0.0)
    xs_ref[...] = x_ref[...] * dinv


def _postscale_body(q_ref, degp_ref, o_ref):
    deg = degp_ref[0] + degp_ref[1]
    dinv = jnp.where(deg > 0.0, lax.rsqrt(deg), 0.0)
    o_ref[...] = (q_ref[0] + q_ref[1]) * dinv


_RB = 1000
_G = N // _RB

_prescale = pl.pallas_call(
    _prescale_body,
    grid=(_G,),
    in_specs=[
        pl.BlockSpec((_RB, D), lambda i: (i, 0)),
        pl.BlockSpec((NC, _RB, 1), lambda i: (0, i, 0)),
    ],
    out_specs=pl.BlockSpec((_RB, D), lambda i: (i, 0)),
    out_shape=jax.ShapeDtypeStruct((N, D), jnp.float32),
)

_postscale = pl.pallas_call(
    _postscale_body,
    grid=(_G,),
    in_specs=[
        pl.BlockSpec((NC, _RB, D), lambda i: (0, i, 0)),
        pl.BlockSpec((NC, _RB, 1), lambda i: (0, i, 0)),
    ],
    out_specs=pl.BlockSpec((_RB, D), lambda i: (i, 0)),
    out_shape=jax.ShapeDtypeStruct((N, D), jnp.float32),
)


@jax.jit
def kernel(x, edge_index):
    src1 = edge_index[0]
    dst3 = edge_index[1].reshape(NW, NCH, K)
    dst4 = edge_index[1].reshape(NW * NCH, 1, K)
    degp = _deg_call(dst3)
    degp3 = degp.reshape(NC, N_PAD, 1)
    xs = _prescale(x, degp3)
    q = _main_call(xs, src1, dst4)
    return _postscale(q, degp3)


# speedup vs baseline: 25.9142x; 1.0809x over previous
import functools

import jax
import jax.numpy as jnp
from jax import lax
from jax.experimental import pallas as pl
from jax.experimental.pallas import tpu as pltpu
from jax.experimental.pallas import tpu_sc as plsc

N = 10000
E = 320000
D = 128
NC = 2
NS = 16
NW = NC * NS
EPT = E // NW
K = 128
PAD = 240
EPT_PAD = EPT + PAD
NCH = EPT_PAD // K
SEG = 640
N_PAD = NS * SEG

_MESH = plsc.VectorSubcoreMesh(core_axis_name="c", subcore_axis_name="s")


def _deg_body(dst3_hbm, out_hbm, dst_idx, buf, acc):
    c = lax.axis_index("c")
    s = lax.axis_index("s")
    wid = s * NC + c
    pltpu.sync_copy(dst3_hbm.at[wid], dst_idx)

    @pl.loop(0, SEG // 16)
    def _(i):
        buf[pl.ds(i * 16, 16)] = jnp.zeros((16,), jnp.float32)

    pltpu.sync_copy(buf, acc.at[pl.ds(s * SEG, SEG)])

    @pl.loop(0, 8)
    def _(i):
        buf[pl.ds(i * 16, 16)] = jnp.ones((16,), jnp.float32)

    plsc.subcore_barrier()

    @pl.loop(0, NCH)
    def _(j):
        pltpu.sync_copy(buf.at[pl.ds(0, K)], acc.at[dst_idx.at[j]], add=True)

    plsc.subcore_barrier()
    pltpu.sync_copy(acc.at[pl.ds(s * SEG, SEG)],
                    out_hbm.at[c].at[pl.ds(s * SEG, SEG)])


_deg_call = functools.partial(
    pl.kernel,
    out_type=jax.ShapeDtypeStruct((NC, N_PAD), jnp.float32),
    mesh=_MESH,
    scratch_types=[
        pltpu.VMEM((NCH, K), jnp.int32),
        pltpu.VMEM((SEG,), jnp.float32),
        pltpu.VMEM_SHARED((N_PAD,), jnp.float32),
    ],
)(_deg_body)


def _main_body(xs_hbm, src4_hbm, dst4_hbm, out_hbm,
               sibuf, sisem, dibuf, disem, rbuf, rsem, acc):
    c = lax.axis_index("c")
    s = lax.axis_index("s")
    wid = s * NC + c
    base = wid * NCH

    @pl.loop(0, K)
    def _(i):
        for jj in range(D // 16):
            rbuf[0][i, pl.ds(jj * 16, 16)] = jnp.zeros((16,), jnp.float32)

    for k in range(SEG // K):
        pltpu.sync_copy(rbuf[0], acc.at[pl.ds(s * SEG + k * K, K)])

    plsc.subcore_barrier()

    def start_idx(jj, slot):
        pltpu.async_copy(src4_hbm.at[base + jj], sibuf[slot], sisem[slot])
        pltpu.async_copy(dst4_hbm.at[base + jj], dibuf[slot], disem[slot])

    def wait_sidx(jj, slot):
        pltpu.make_async_copy(
            src4_hbm.at[base + jj], sibuf[slot], sisem[slot]).wait()

    def wait_didx(jj, slot):
        pltpu.make_async_copy(
            dst4_hbm.at[base + jj], dibuf[slot], disem[slot]).wait()

    def start_rows(slot, rb):
        pltpu.async_copy(xs_hbm.at[sibuf[slot].at[0]], rbuf[rb], rsem[rb])

    def wait_rows(slot, rb):
        pltpu.make_async_copy(
            xs_hbm.at[sibuf[slot].at[0]], rbuf[rb], rsem[rb]).wait()

    for t in range(4):
        start_idx(t, t)
    for t in range(2):
        wait_sidx(t, t)
        start_rows(t, t)

    @pl.loop(0, NCH, step=4)
    def _(j):
        for b in range(4):
            jj = j + b
            rb = b % 2
            wait_rows(b, rb)
            wait_didx(jj, b)
            pltpu.sync_copy(rbuf[rb], acc.at[dibuf[b].at[0]], add=True)

            @pl.when(jj + 4 < NCH)
            def _():
                start_idx(jj + 4, b)

            @pl.when(jj + 2 < NCH)
            def _():
                wait_sidx(jj + 2, (b + 2) % 4)
                start_rows((b + 2) % 4, rb)

    plsc.subcore_barrier()
    pltpu.sync_copy(acc.at[pl.ds(s * SEG, SEG)],
                    out_hbm.at[c].at[pl.ds(s * SEG, SEG)])


_main_call = functools.partial(
    pl.kernel,
    out_type=jax.ShapeDtypeStruct((NC, N_PAD, D), jnp.float32),
    mesh=_MESH,
    scratch_types=[
        [pltpu.VMEM((1, K), jnp.int32)] * 4,
        [pltpu.SemaphoreType.DMA] * 4,
        [pltpu.VMEM((1, K), jnp.int32)] * 4,
        [pltpu.SemaphoreType.DMA] * 4,
        [pltpu.VMEM((K, D), jnp.float32)] * 2,
        [pltpu.SemaphoreType.DMA] * 2,
        pltpu.VMEM_SHARED((N_PAD, D), jnp.float32),
    ],
)(_main_body)


def _prescale_body(x_ref, degp_ref, xs_ref):
    deg = degp_ref[0] + degp_ref[1]
    dinv = jnp.where(deg > 0.0, lax.rsqrt(deg), 0.0)
    xs_ref[...] = x_ref[...] * dinv


def _postscale_body(q_ref, degp_ref, o_ref):
    deg = degp_ref[0] + degp_ref[1]
    dinv = jnp.where(deg > 0.0, lax.rsqrt(deg), 0.0)
    o_ref[...] = (q_ref[0] + q_ref[1]) * dinv


_RB = 1000
_G = N // _RB

_prescale = pl.pallas_call(
    _prescale_body,
    grid=(_G,),
    in_specs=[
        pl.BlockSpec((_RB, D), lambda i: (i, 0)),
        pl.BlockSpec((NC, _RB, 1), lambda i: (0, i, 0)),
    ],
    out_specs=pl.BlockSpec((_RB, D), lambda i: (i, 0)),
    out_shape=jax.ShapeDtypeStruct((N, D), jnp.float32),
)

_postscale = pl.pallas_call(
    _postscale_body,
    grid=(_G,),
    in_specs=[
        pl.BlockSpec((NC, _RB, D), lambda i: (0, i, 0)),
        pl.BlockSpec((NC, _RB, 1), lambda i: (0, i, 0)),
    ],
    out_specs=pl.BlockSpec((_RB, D), lambda i: (i, 0)),
    out_shape=jax.ShapeDtypeStruct((N, D), jnp.float32),
)


@jax.jit
def kernel(x, edge_index):
    srcm = edge_index[0].reshape(NW, EPT)
    dstm = edge_index[1].reshape(NW, EPT)
    pad_src = jnp.broadcast_to((jnp.arange(PAD, dtype=jnp.int32) * 41) % N,
                               (NW, PAD))
    pad_dst = jnp.broadcast_to(N + jnp.arange(PAD, dtype=jnp.int32),
                               (NW, PAD))
    srcp = jnp.concatenate([srcm, pad_src], axis=1)
    dstp = jnp.concatenate([dstm, pad_dst], axis=1)
    src4 = srcp.reshape(NW * NCH, 1, K)
    dst4 = dstp.reshape(NW * NCH, 1, K)
    dst3 = dstp.reshape(NW, NCH, K)
    degp = _deg_call(dst3)
    degp3 = degp.reshape(NC, N_PAD, 1)
    xs = _prescale(x, degp3)
    q = _main_call(xs, src4, dst4)
    return _postscale(q, degp3)


# speedup vs baseline: 26.9672x; 1.0406x over previous
import functools

import jax
import jax.numpy as jnp
from jax import lax
from jax.experimental import pallas as pl
from jax.experimental.pallas import tpu as pltpu
from jax.experimental.pallas import tpu_sc as plsc

N = 10000
E = 320000
D = 128
NC = 2
NS = 16
NW = NC * NS
K = 128
GCH = E // K
NCH = GCH // NW
NCH_MAX = NCH + 2
DCH = GCH // NS
DCH_MAX = DCH + 4
SEG = 640
N_PAD = NS * SEG
HSEG = SEG // 2

_MESH = plsc.VectorSubcoreMesh(core_axis_name="c", subcore_axis_name="s")


def _dinv_body(edge_hbm, out_hbm, dibuf, disem, ones, dv, bc, acc):
    c = lax.axis_index("c")
    s = lax.axis_index("s")
    nch = DCH + jnp.where(s < GCH - DCH * NS, 1, 0)

    @pl.loop(0, SEG // 16)
    def _(i):
        ones[pl.ds(i * 16, 16)] = jnp.zeros((16,), jnp.float32)

    pltpu.sync_copy(ones, acc.at[pl.ds(s * SEG, SEG)])

    @pl.loop(0, K // 16)
    def _(i):
        ones[pl.ds(i * 16, 16)] = jnp.ones((16,), jnp.float32)

    plsc.subcore_barrier()

    def start_idx(jj, slot):
        g = jj * NS + s
        pltpu.async_copy(edge_hbm.at[1, pl.ds(g * K, K)], dibuf[slot],
                         disem[slot])

    def wait_idx(slot):
        pltpu.make_async_copy(edge_hbm.at[1, pl.ds(0, K)], dibuf[slot],
                              disem[slot]).wait()

    for t in range(4):
        @pl.when(t < nch)
        def _():
            start_idx(t, t)

    @pl.loop(0, DCH_MAX, step=4)
    def _(j):
        for b in range(4):
            jj = j + b

            @pl.when(jj < nch)
            def _():
                wait_idx(b)
                pltpu.sync_copy(ones.at[pl.ds(0, K)], acc.at[dibuf[b]],
                                add=True)

            @pl.when(jj + 4 < nch)
            def _():
                start_idx(jj + 4, b)

    plsc.subcore_barrier()

    base = c * (N_PAD // 2) + s * HSEG
    pltpu.sync_copy(acc.at[pl.ds(base, HSEG)], dv)

    @pl.loop(0, HSEG // 16)
    def _(i):
        v = dv[pl.ds(i * 16, 16)]
        bits = plsc.bitcast(v, jnp.int32)
        bits = jnp.int32(0x5F3759DF) - (bits >> 1)
        y = plsc.bitcast(bits, jnp.float32)
        for _ in range(3):
            y = y * (1.5 - 0.5 * v * y * y)
        dv[pl.ds(i * 16, 16)] = jnp.where(v > 0.5, y, 0.0)

    @pl.loop(0, HSEG)
    def _(r):
        idxv = jnp.full((16,), r, dtype=jnp.int32)
        row = plsc.load_gather(dv, [idxv])
        for jj in range(D // 16):
            bc[r, pl.ds(jj * 16, 16)] = row

    pltpu.sync_copy(bc, out_hbm.at[pl.ds(base, HSEG)])


_dinv_call = functools.partial(
    pl.kernel,
    out_type=jax.ShapeDtypeStruct((N_PAD, D), jnp.float32),
    mesh=_MESH,
    scratch_types=[
        [pltpu.VMEM((K,), jnp.int32)] * 4,
        [pltpu.SemaphoreType.DMA] * 4,
        pltpu.VMEM((SEG,), jnp.float32),
        pltpu.VMEM((HSEG,), jnp.float32),
        pltpu.VMEM((HSEG, D), jnp.float32),
        pltpu.VMEM_SHARED((N_PAD,), jnp.float32),
    ],
    compiler_params=pltpu.CompilerParams(needs_layout_passes=False),
)(_dinv_body)


def _main_body(xs_hbm, edge_hbm, out_hbm,
               sibuf, sisem, dibuf, disem, rbuf, rsem, acc):
    c = lax.axis_index("c")
    s = lax.axis_index("s")
    wid = s * NC + c
    nch = NCH + jnp.where(wid < GCH - NCH * NW, 1, 0)

    def chunk(jj):
        return jnp.where(jj < NCH, wid * NCH + jj, NCH * NW + wid)

    @pl.loop(0, K)
    def _(i):
        for jj in range(D // 16):
            rbuf[0][i, pl.ds(jj * 16, 16)] = jnp.zeros((16,), jnp.float32)

    for k in range(SEG // K):
        pltpu.sync_copy(rbuf[0], acc.at[pl.ds(s * SEG + k * K, K)])

    plsc.subcore_barrier()

    def start_idx(jj, slot):
        g = chunk(jj)
        pltpu.async_copy(edge_hbm.at[0, pl.ds(g * K, K)], sibuf[slot],
                         sisem[slot])
        pltpu.async_copy(edge_hbm.at[1, pl.ds(g * K, K)], dibuf[slot],
                         disem[slot])

    def wait_sidx(slot):
        pltpu.make_async_copy(edge_hbm.at[0, pl.ds(0, K)], sibuf[slot],
                              sisem[slot]).wait()

    def wait_didx(slot):
        pltpu.make_async_copy(edge_hbm.at[1, pl.ds(0, K)], dibuf[slot],
                              disem[slot]).wait()

    def start_rows(slot, rb):
        pltpu.async_copy(xs_hbm.at[sibuf[slot]], rbuf[rb], rsem[rb])

    def wait_rows(slot, rb):
        pltpu.make_async_copy(xs_hbm.at[sibuf[slot]], rbuf[rb],
                              rsem[rb]).wait()

    for t in range(4):
        start_idx(t, t)
    for t in range(2):
        wait_sidx(t)
        start_rows(t, t)

    @pl.loop(0, NCH_MAX, step=4)
    def _(j):
        for b in range(4):
            jj = j + b
            rb = b % 2

            @pl.when(jj < nch)
            def _():
                wait_rows(b, rb)
                wait_didx(b)
                pltpu.sync_copy(rbuf[rb], acc.at[dibuf[b]], add=True)

            @pl.when(jj + 4 < nch)
            def _():
                start_idx(jj + 4, b)

            @pl.when(jj + 2 < nch)
            def _():
                wait_sidx((b + 2) % 4)
                start_rows((b + 2) % 4, rb)

    plsc.subcore_barrier()
    pltpu.sync_copy(acc.at[pl.ds(s * SEG, SEG)],
                    out_hbm.at[c].at[pl.ds(s * SEG, SEG)])


_main_call = functools.partial(
    pl.kernel,
    out_type=jax.ShapeDtypeStruct((NC, N_PAD, D), jnp.float32),
    mesh=_MESH,
    scratch_types=[
        [pltpu.VMEM((K,), jnp.int32)] * 4,
        [pltpu.SemaphoreType.DMA] * 4,
        [pltpu.VMEM((K,), jnp.int32)] * 4,
        [pltpu.SemaphoreType.DMA] * 4,
        [pltpu.VMEM((K, D), jnp.float32)] * 2,
        [pltpu.SemaphoreType.DMA] * 2,
        pltpu.VMEM_SHARED((N_PAD, D), jnp.float32),
    ],
)(_main_body)


def _prescale_body(x_ref, dinv_ref, xs_ref):
    xs_ref[...] = x_ref[...] * dinv_ref[...]


def _postscale_body(q_ref, dinv_ref, o_ref):
    o_ref[...] = (q_ref[0] + q_ref[1]) * dinv_ref[...]


_RB = 1000
_G = N // _RB

_prescale = pl.pallas_call(
    _prescale_body,
    grid=(_G,),
    in_specs=[
        pl.BlockSpec((_RB, D), lambda i: (i, 0)),
        pl.BlockSpec((_RB, D), lambda i: (i, 0)),
    ],
    out_specs=pl.BlockSpec((_RB, D), lambda i: (i, 0)),
    out_shape=jax.ShapeDtypeStruct((N, D), jnp.float32),
)

_postscale = pl.pallas_call(
    _postscale_body,
    grid=(_G,),
    in_specs=[
        pl.BlockSpec((NC, _RB, D), lambda i: (0, i, 0)),
        pl.BlockSpec((_RB, D), lambda i: (i, 0)),
    ],
    out_specs=pl.BlockSpec((_RB, D), lambda i: (i, 0)),
    out_shape=jax.ShapeDtypeStruct((N, D), jnp.float32),
)


@jax.jit
def kernel(x, edge_index):
    dinv = _dinv_call(edge_index)
    xs = _prescale(x, dinv)
    q = _main_call(xs, edge_index)
    return _postscale(q, dinv)
